# baseline (speedup 1.0000x reference)
import functools
import math

import jax
import jax.numpy as jnp
from jax import lax
from jax.experimental import pallas as pl
from jax.experimental.pallas import tpu as pltpu
from jax.experimental.pallas import tpu_sc as plsc

D_MODEL = 64
SCALE = math.sqrt(D_MODEL)

NC = 2
NS = 16
NW = NC * NS
L = 16
CW = 128
NBUF = 4


@functools.partial(jax.jit, static_argnums=(2, 3))
def _emb_lookup(lut, x2, nchunk, nb):
    mesh = plsc.VectorSubcoreMesh(
        core_axis_name="c", subcore_axis_name="s", num_cores=NC, num_subcores=NS
    )
    assert (nchunk - (NBUF - 1) - 1) % NBUF == 0

    def body(lut_h, x_h, out_h, idx_v, rb0, rb1, rb2, rb3,
             gs0, gs1, gs2, gs3, ss0, ss1, ss2, ss3):
        rb = [rb0, rb1, rb2, rb3]
        gs = [gs0, gs1, gs2, gs3]
        ss = [ss0, ss1, ss2, ss3]
        wid = lax.axis_index("s") * NC + lax.axis_index("c")
        base = wid * nb
        pltpu.sync_copy(x_h.at[wid], idx_v)

        def start_gather(g, b):
            pltpu.async_copy(lut_h.at[idx_v.at[g]], rb[b], gs[b])

        def wait_gather(g, b):
            pltpu.make_async_copy(lut_h.at[idx_v.at[g]], rb[b], gs[b]).wait()

        def scale(b):
            def srow(r, c):
                for j in range(D_MODEL // L):
                    sl = pl.ds(j * L, L)
                    rb[b][r, sl] = rb[b][r, sl] * SCALE
                return c
            lax.fori_loop(0, CW, srow, 0, unroll=4)

        def start_scatter(g, b):
            pltpu.async_copy(rb[b], out_h.at[pl.ds(base + g * CW, CW)], ss[b])

        def wait_scatter(b):
            pltpu.make_async_copy(rb[b], out_h.at[pl.ds(base, CW)], ss[b]).wait()

        for b in range(NBUF - 1):
            start_gather(b, b)
        start_gather(NBUF - 1, NBUF - 1)
        wait_gather(0, 0)
        scale(0)
        start_scatter(0, 0)

        def block(gg, carry):
            g0 = 1 + gg * NBUF
            for b in range(NBUF):
                g = g0 + b
                bb = (1 + b) % NBUF
                bn = b % NBUF
                wait_scatter(bn)
                start_gather(g + NBUF - 1, bn)
                wait_gather(g, bb)
                scale(bb)
                start_scatter(g, bb)
            return carry

        nblocks = (nchunk - NBUF) // NBUF
        lax.fori_loop(0, nblocks, block, 0)

        for g in range(nchunk - NBUF + 1, nchunk):
            b = g % NBUF
            wait_gather(g, b)
            scale(b)
            start_scatter(g, b)
        for b in range(NBUF):
            wait_scatter(b)

    B = nb * NW
    f = pl.kernel(
        body,
        out_type=jax.ShapeDtypeStruct((B, D_MODEL), jnp.float32),
        mesh=mesh,
        compiler_params=pltpu.CompilerParams(use_tc_tiling_on_sc=False),
        scratch_types=[
            pltpu.VMEM((nchunk, CW), jnp.int32),
            pltpu.VMEM((CW, D_MODEL), jnp.float32),
            pltpu.VMEM((CW, D_MODEL), jnp.float32),
            pltpu.VMEM((CW, D_MODEL), jnp.float32),
            pltpu.VMEM((CW, D_MODEL), jnp.float32),
            pltpu.SemaphoreType.DMA,
            pltpu.SemaphoreType.DMA,
            pltpu.SemaphoreType.DMA,
            pltpu.SemaphoreType.DMA,
            pltpu.SemaphoreType.DMA,
            pltpu.SemaphoreType.DMA,
            pltpu.SemaphoreType.DMA,
            pltpu.SemaphoreType.DMA,
        ],
    )
    return f(lut, x2)


def kernel(x, lut):
    B = x.size
    nb = B // NW
    nchunk = nb // CW
    x2 = x.reshape(NW, nchunk, CW)
    out = _emb_lookup(lut, x2, nchunk, nb)
    return out.reshape(*x.shape, D_MODEL)
